# baseline (speedup 1.0000x reference)
import functools

import jax
import jax.numpy as jnp
from jax import lax
from jax.experimental import pallas as pl
from jax.experimental.pallas import tpu as pltpu
from jax.experimental.pallas import tpu_sc as plsc

_PROB = 0.12
_LANES = 16
_UNROLL = 8


def _final_indices(T):
    rkey = jax.random.key(42)
    k1, k2 = jax.random.split(rkey)
    replace = jax.random.uniform(k1, (T,)) < _PROB
    direction = jnp.where(jax.random.uniform(k2, (T,)) < 0.5, -1, 1)
    idx = jnp.arange(T)
    offset = jnp.where(idx == 0, 1, jnp.where(idx == T - 1, -1, direction))
    return jnp.where(replace, idx + offset, idx).astype(jnp.int32)


def _make_sc_permute(R, T, rb):
    info = plsc.get_sparse_core_info()
    nw = info.num_cores * info.num_subcores
    rows_per_w = R // nw
    nb = rows_per_w // rb
    chunks = T // _LANES
    mesh = plsc.VectorSubcoreMesh(core_axis_name="c", subcore_axis_name="s")

    @functools.partial(
        pl.kernel,
        out_type=jax.ShapeDtypeStruct((R, T), jnp.float32),
        mesh=mesh,
        compiler_params=pltpu.CompilerParams(needs_layout_passes=False),
        scratch_types=[
            pltpu.VMEM((T,), jnp.int32),
            pltpu.VMEM((4, rb, T), jnp.float32),
            pltpu.VMEM((2, rb, T), jnp.float32),
            pltpu.SemaphoreType.DMA,
            pltpu.SemaphoreType.DMA,
            pltpu.SemaphoreType.DMA,
            pltpu.SemaphoreType.DMA,
            pltpu.SemaphoreType.DMA,
            pltpu.SemaphoreType.DMA,
        ],
    )
    def k(x_hbm, fidx_hbm, out_hbm, fidx_v, inb, outb,
          si0, si1, si2, si3, so0, so1):
        wid = lax.axis_index("s") * info.num_cores + lax.axis_index("c")
        row_base = wid * rows_per_w
        sin = (si0, si1, si2, si3)
        sout = (so0, so1)

        def in_sl(b):
            return x_hbm.at[pl.ds(row_base + b * rb, rb)]

        def out_sl(b):
            return out_hbm.at[pl.ds(row_base + b * rb, rb)]

        for p in range(4):
            pltpu.async_copy(in_sl(p), inb.at[p], sin[p])
        pltpu.sync_copy(fidx_hbm, fidx_v)

        def outer(bb, _):
            for buf in range(4):
                b = bb * 4 + buf
                obuf = buf % 2
                pltpu.make_async_copy(in_sl(b), inb.at[buf], sin[buf]).wait()

                @pl.when(b >= 2)
                def _():
                    pltpu.make_async_copy(
                        outb.at[obuf], out_sl(b - 2), sout[obuf]
                    ).wait()

                bufv = jnp.full((_LANES,), buf, jnp.int32)

                @plsc.parallel_loop(
                    0, chunks * _LANES, _LANES, unroll=_UNROLL
                )
                def _(i):
                    sl = pl.ds(i, _LANES)
                    idxv = fidx_v[sl]
                    for r in range(rb):
                        rowv = jnp.full((_LANES,), r, jnp.int32)
                        outb[obuf, r, sl] = plsc.load_gather(
                            inb, [bufv, rowv, idxv]
                        )

                pltpu.async_copy(outb.at[obuf], out_sl(b), sout[obuf])

                @pl.when(b + 4 < nb)
                def _():
                    pltpu.async_copy(in_sl(b + 4), inb.at[buf], sin[buf])

            return 0

        lax.fori_loop(0, nb // 4, outer, 0)
        pltpu.make_async_copy(outb.at[0], out_sl(nb - 2), sout[0]).wait()
        pltpu.make_async_copy(outb.at[1], out_sl(nb - 1), sout[1]).wait()

    return k


def kernel(quantized):
    B, C, T = quantized.shape
    R = B * C
    x = quantized.reshape(R, T)
    fidx = _final_indices(T)
    out = _make_sc_permute(R, T, rb=4)(x, fidx)
    return out.reshape(B, C, T)
